# baseline (speedup 1.0000x reference)
import functools

import jax
import jax.numpy as jnp
from jax import lax
from jax.experimental import pallas as pl
from jax.experimental.pallas import tpu as pltpu
from jax.experimental.pallas import tpu_sc as plsc

B = 16384
J_IN = 144
J_OUT = 118
C = 3
W = 256
MAP_PAD = 128

NUM_WORKERS = 32
N_CHUNKS = B // W
N_UNITS = C * N_CHUNKS
UNITS_PER_W = N_UNITS // NUM_WORKERS
N_BUF = 3


def _sc_rowgather(tin, ridx_padded):
    mesh = plsc.VectorSubcoreMesh(core_axis_name="c", subcore_axis_name="s")

    @functools.partial(
        pl.kernel,
        out_type=jax.ShapeDtypeStruct((C, J_OUT, B), jnp.float32),
        mesh=mesh,
        compiler_params=pltpu.CompilerParams(needs_layout_passes=False),
        scratch_types=[
            pltpu.VMEM((C, MAP_PAD), jnp.int32),
            pltpu.VMEM((J_OUT, W), jnp.float32),
            pltpu.VMEM((J_OUT, W), jnp.float32),
            pltpu.VMEM((J_OUT, W), jnp.float32),
            pltpu.SemaphoreType.DMA,
            pltpu.SemaphoreType.DMA,
        ],
    )
    def k(in_hbm, map_hbm, out_hbm, map_v, g0, g1, g2, sem_g, sem_o):
        wid = lax.axis_index("s") * 2 + lax.axis_index("c")
        pltpu.sync_copy(map_hbm, map_v)
        gbuf = (g0, g1, g2)

        def unit_cw(u):
            uid = wid + NUM_WORKERS * u
            return uid // N_CHUNKS, (uid % N_CHUNKS) * W

        def start_gather(u):
            c, w0 = unit_cw(u)
            return pltpu.async_copy(
                in_hbm.at[map_v.at[c, pl.ds(0, J_OUT)], pl.ds(w0, W)],
                gbuf[u % N_BUF],
                sem_g,
            )

        def start_out(u):
            c, w0 = unit_cw(u)
            return pltpu.async_copy(
                gbuf[u % N_BUF], out_hbm.at[c, :, pl.ds(w0, W)], sem_o
            )

        d_g = {0: start_gather(0)}
        d_out = {}
        for u in range(UNITS_PER_W):
            if u >= 2:
                d_out[u - 2].wait()
            if u + 1 < UNITS_PER_W:
                d_g[u + 1] = start_gather(u + 1)
            d_g[u].wait()
            d_out[u] = start_out(u)
        d_out[UNITS_PER_W - 2].wait()
        d_out[UNITS_PER_W - 1].wait()

    return k(tin, ridx_padded)


def kernel(joints, joint_maps):
    tin = jnp.transpose(joints, (2, 1, 0)).reshape(C * J_IN, B)
    ridx = joint_maps.astype(jnp.int32)[None, :] + (
        jnp.arange(C, dtype=jnp.int32) * J_IN
    )[:, None]
    ridx_padded = jnp.zeros((C, MAP_PAD), jnp.int32).at[:, :J_OUT].set(ridx)
    tout = _sc_rowgather(tin, ridx_padded)
    return jnp.transpose(tout, (2, 1, 0))
